# baseline (speedup 1.0000x reference)
import functools

import jax
import jax.numpy as jnp
from jax import lax
from jax.experimental import pallas as pl
from jax.experimental.pallas import tpu as pltpu
from jax.experimental.pallas import tpu_sc as plsc

N_IN = 25000
N_OUT = 50000
NNZ = 200000
C_IN = 128
C_OUT = 32
L = 9

CC = 32
NQ = 4
BATCH = 128
N_PAD = 51200
BATCH_A = 64
NB_A = 200
NNZ_PAD = 204800
CHK = 8
NCHK = NB_A // CHK
ROWS_PER_TILE = N_PAD // 16
ZROWS = 200
N_PAD_C = 50048
NB_C = N_PAD_C // BATCH
MM_BLK = 2048
NG = 3

_SPLAT_DNUMS = lax.GatherDimensionNumbers(
    offset_dims=(), collapsed_slice_dims=(0,), start_index_map=(0,))


def _lane_splat(vec16, lane):
    idx = jnp.full((16, 1), lane, jnp.int32)
    return lax.gather(vec16, idx, dimension_numbers=_SPLAT_DNUMS,
                      slice_sizes=(1,),
                      mode=lax.GatherScatterMode.PROMISE_IN_BOUNDS)


def _pool_body(xf, cqa, rowa, vala, pooled,
               spmem, rows0, rows1, rows2, rows3,
               rows4, rows5, rows6, rows7,
               colb0, colb1, rowb0, rowb1, valb0, valb1, rowg, zbuf,
               semg0, semg1, semg2, semg3, semg4, semg5, semg6, semg7,
               sems0, sems1, sems2, sems3, sems4, sems5, sems6, sems7,
               semi, semz):
    c = lax.axis_index("c")
    t = lax.axis_index("s")
    rows = (rows0, rows1, rows2, rows3, rows4, rows5, rows6, rows7)
    semg = (semg0, semg1, semg2, semg3, semg4, semg5, semg6, semg7)
    sems = (sems0, sems1, sems2, sems3, sems4, sems5, sems6, sems7)
    colb = (colb0, colb1)
    rowb = (rowb0, rowb1)
    valb = (valb0, valb1)

    zero16 = jnp.zeros((16,), jnp.float32)

    def _zb(i, carry):
        zbuf[i, pl.ds(0, 16)] = zero16
        zbuf[i, pl.ds(16, 16)] = zero16
        return carry

    lax.fori_loop(0, ZROWS, _zb, 0)

    def _idx_fire(ib, q, ck):
        pltpu.async_copy(cqa.at[q, t, pl.ds(ck * CHK, CHK)], colb[ib], semi)
        pltpu.async_copy(rowa.at[t, pl.ds(ck * CHK, CHK)], rowb[ib], semi)
        pltpu.async_copy(vala.at[t, pl.ds(ck * CHK, CHK)], valb[ib], semi)

    def _idx_wait(ib, q):
        pltpu.make_async_copy(cqa.at[q, t, pl.ds(0, CHK)],
                              colb[ib], semi).wait()
        pltpu.make_async_copy(rowa.at[t, pl.ds(0, CHK)],
                              rowb[ib], semi).wait()
        pltpu.make_async_copy(vala.at[t, pl.ds(0, CHK)],
                              valb[ib], semi).wait()

    def _scale(rref, vref, j):
        def _grp(g, carry):
            vv = vref[j, pl.ds(g * 16, 16)]
            for i16 in range(16):
                v = _lane_splat(vv, i16)
                i = g * 16 + i16
                rref[i, pl.ds(0, 16)] = rref[i, pl.ds(0, 16)] * v
                rref[i, pl.ds(16, 16)] = rref[i, pl.ds(16, 16)] * v
            return carry

        lax.fori_loop(0, 4, _grp, 0)

    def _gfire(rr, ib, j):
        pltpu.async_copy(xf.at[colb[ib].at[j]], rows[rr], semg[rr])

    def _gwait(r):
        pltpu.make_async_copy(xf.at[colb[0].at[0]], rows[r], semg[r]).wait()

    def _sfire(r, ib, j):
        def _rg(g, carry):
            sl = pl.ds(g * 16, 16)
            rowg[r, sl] = rowb[ib][j, sl]
            return carry

        lax.fori_loop(0, 4, _rg, 0)
        pltpu.async_copy(rows[r], spmem.at[rowg.at[r]], sems[r], add=True)

    def _swait(r):
        pltpu.make_async_copy(rows[r], spmem.at[rowg.at[r]], sems[r]).wait()

    def _chunk(n, ib_cur, ib_nxt, q, first=False, last=False):
        for r in range(CHK):
            _gwait(r)
            _scale(rows[r], valb[ib_cur], r)
            _sfire(r, ib_cur, r)
            if r == 0 and not last:
                _idx_fire(ib_nxt, q, n + 1)
            if r == 4 and not last:
                _idx_wait(ib_nxt, q)
            rr = (r + 4) % 8
            if not (first and r < 4):
                _swait(rr)
            if not (last and r >= 4):
                if r < 4:
                    _gfire(rr, ib_cur, r + 4)
                else:
                    _gfire(rr, ib_nxt, r - 4)

    def _pass(p, carry):
        q = 2 * c + p
        zd = [pltpu.async_copy(
            zbuf, spmem.at[pl.ds(t * ROWS_PER_TILE + z * ZROWS, ZROWS)], semz)
            for z in range(ROWS_PER_TILE // ZROWS)]
        for d in zd:
            d.wait()
        plsc.subcore_barrier()

        _idx_fire(0, q, 0)
        _idx_wait(0, q)
        for r in range(4):
            _gfire(r, 0, r)
        _chunk(0, 0, 1, q, first=True)

        def _two(m, carry2):
            _chunk(1 + 2 * m, 1, 0, q)
            _chunk(2 + 2 * m, 0, 1, q)
            return carry2

        lax.fori_loop(0, (NCHK - 3) // 2, _two, 0)

        _chunk(NCHK - 2, 1, 0, q)
        _chunk(NCHK - 1, 0, 1, q, last=True)
        for r in range(4, 8):
            _swait(r)

        plsc.subcore_barrier()

        sl = pl.ds(t * ROWS_PER_TILE, ROWS_PER_TILE)
        pltpu.sync_copy(spmem.at[sl], pooled.at[sl, pl.ds(CC * q, CC)])
        plsc.subcore_barrier()
        return carry

    lax.fori_loop(0, 2, _pass, 0)


def _matmul_body(xr, w, *outs):
    acc = lax.dot_general(xr[...], w[...], (((1,), (0,)), ((), ())),
                          preferred_element_type=jnp.float32)
    for g in range(NG):
        outs[g][...] = acc[:, C_IN * g:C_IN * (g + 1)]


def _spiral_body(st3, yf0, yf1, yf2, bias, out,
                 idxa, idxb,
                 ga0, ga1, ga2, ga3, ga4, ga5, ga6, ga7, ga8,
                 gb0, gb1, gb2, gb3, gb4, gb5, gb6, gb7, gb8,
                 obuf, bvm, sema, semb):
    c = lax.axis_index("c")
    s = lax.axis_index("s")
    wid = s * 2 + c
    yfs = (yf0, yf1, yf2)
    ga = (ga0, ga1, ga2, ga3, ga4, ga5, ga6, ga7, ga8)
    gb = (gb0, gb1, gb2, gb3, gb4, gb5, gb6, gb7, gb8)

    pltpu.sync_copy(bias, bvm)
    b0 = bvm[pl.ds(0, 16)]
    b1 = bvm[pl.ds(16, 16)]

    def _fire(kb, idx, g, sem):
        pltpu.sync_copy(st3.at[kb], idx)
        for j in range(L):
            def _xf(gg, carry, j=j):
                sl = pl.ds(gg * 16, 16)
                idx[j, sl] = idx[j, sl] * 4 + (j % 4)
                return carry

            lax.fori_loop(0, 8, _xf, 0)
        return [pltpu.async_copy(yfs[j // 4].at[idx.at[j]], g[j], sem)
                for j in range(L)]

    def _compute_store(kb, g):
        def _node(i, ncarry):
            for k in range(2):
                sl = pl.ds(16 * k, 16)
                acc = g[0][i, sl]
                for j in range(1, L):
                    acc = acc + g[j][i, sl]
                acc = acc + (b0 if k == 0 else b1)
                e = jnp.exp(acc) - 1.0
                obuf[i, sl] = jnp.where(acc > 0.0, acc, e)
            return ncarry

        lax.fori_loop(0, BATCH, _node, 0)
        pltpu.sync_copy(obuf, out.at[pl.ds(kb * BATCH, BATCH)])

    count = (NB_C - wid + 31) // 32
    cnt2 = count // 2

    def _pair(m, carry):
        kb_a = wid + (2 * m) * 32
        kb_b = wid + (2 * m + 1) * 32
        da = _fire(kb_a, idxa, ga, sema)
        db = _fire(kb_b, idxb, gb, semb)
        for d in da:
            d.wait()
        _compute_store(kb_a, ga)
        for d in db:
            d.wait()
        _compute_store(kb_b, gb)
        return carry

    lax.fori_loop(0, cnt2, _pair, 0)

    @pl.when(count % 2 == 1)
    def _tail():
        kb = wid + (2 * cnt2) * 32
        da = _fire(kb, idxa, ga, sema)
        for d in da:
            d.wait()
        _compute_store(kb, ga)


def _pool_call(xf, cqa, rowa, vala):
    mesh = plsc.VectorSubcoreMesh(core_axis_name="c", subcore_axis_name="s")
    f = pl.kernel(
        _pool_body,
        out_type=jax.ShapeDtypeStruct((N_PAD, C_IN), jnp.float32),
        mesh=mesh,
        scratch_types=(
            [pltpu.VMEM_SHARED((N_PAD, CC), jnp.float32)]
            + [pltpu.VMEM((BATCH_A, CC), jnp.float32) for _ in range(8)]
            + [pltpu.VMEM((CHK, BATCH_A), jnp.int32) for _ in range(4)]
            + [pltpu.VMEM((CHK, BATCH_A), jnp.float32) for _ in range(2)]
            + [
                pltpu.VMEM((8, BATCH_A), jnp.int32),
                pltpu.VMEM((ZROWS, CC), jnp.float32),
            ]
            + [pltpu.SemaphoreType.DMA for _ in range(18)]
        ),
        compiler_params=pltpu.CompilerParams(use_tc_tiling_on_sc=False),
    )
    return f(xf, cqa, rowa, vala)


def _matmul_call(pooled, w_pad):
    grid = (N_PAD // MM_BLK,)
    in_specs = [pl.BlockSpec((MM_BLK, C_IN), lambda i: (i, 0)),
                pl.BlockSpec((C_IN, NG * C_IN), lambda i: (0, 0))]
    out_specs = [pl.BlockSpec((MM_BLK, C_IN), lambda i: (i, 0))
                 for _ in range(NG)]
    return pl.pallas_call(
        _matmul_body,
        grid=grid,
        in_specs=in_specs,
        out_specs=out_specs,
        out_shape=[jax.ShapeDtypeStruct((N_PAD, C_IN), jnp.float32)
                   for _ in range(NG)],
    )(pooled, w_pad)


def _spiral_call(st3, yfs, bias):
    mesh = plsc.VectorSubcoreMesh(core_axis_name="c", subcore_axis_name="s")
    f = pl.kernel(
        _spiral_body,
        out_type=jax.ShapeDtypeStruct((N_PAD_C, C_OUT), jnp.float32),
        mesh=mesh,
        scratch_types=(
            [pltpu.VMEM((L, BATCH), jnp.int32) for _ in range(2)]
            + [pltpu.VMEM((BATCH, C_OUT), jnp.float32) for _ in range(2 * L)]
            + [
                pltpu.VMEM((BATCH, C_OUT), jnp.float32),
                pltpu.VMEM((C_OUT,), jnp.float32),
                pltpu.SemaphoreType.DMA,
                pltpu.SemaphoreType.DMA,
            ]
        ),
        compiler_params=pltpu.CompilerParams(use_tc_tiling_on_sc=False),
    )
    return f(st3, *yfs, bias)


def kernel(x, trans_row, trans_col, trans_val, spiral_indices, W, b):
    xf = x.reshape(N_IN * NQ, CC)

    pad = NNZ_PAD - NNZ
    rowa = jnp.pad(trans_row.astype(jnp.int32), (0, pad)).reshape(
        16, NB_A, BATCH_A)
    colp2 = jnp.pad(trans_col.astype(jnp.int32), (0, pad)).reshape(
        16, NB_A, BATCH_A)
    vala = jnp.pad(trans_val, (0, pad)).reshape(16, NB_A, BATCH_A)
    cqa = colp2[None] * 4 + jnp.arange(4, dtype=jnp.int32).reshape(4, 1, 1, 1)

    st3 = jnp.pad(spiral_indices.astype(jnp.int32),
                  ((0, N_PAD_C - N_OUT), (0, 0)))
    st3 = st3.T.reshape(L, NB_C, BATCH).transpose(1, 0, 2)

    w_perm = W.reshape(L, C_IN, C_OUT).transpose(1, 0, 2).reshape(
        C_IN, L * C_OUT)
    w_pad = jnp.pad(w_perm, ((0, 0), (0, NG * C_IN - L * C_OUT)))

    pooled = _pool_call(xf, cqa, rowa, vala)
    ys = _matmul_call(pooled, w_pad)
    yfs = [y.reshape(NQ * N_PAD, CC) for y in ys]
    out = _spiral_call(st3, yfs, b)

    return out[:N_OUT].reshape(1, N_OUT, C_OUT)


# speedup vs baseline: 1.1723x; 1.1723x over previous
import functools

import jax
import jax.numpy as jnp
from jax import lax
from jax.experimental import pallas as pl
from jax.experimental.pallas import tpu as pltpu
from jax.experimental.pallas import tpu_sc as plsc

N_IN = 25000
N_OUT = 50000
NNZ = 200000
C_IN = 128
C_OUT = 32
L = 9

CC = 32
NQ = 4
BATCH = 128
N_PAD = 51200
NNZ_PAD = 200704
NB_A = 98
BPC = 7
NCHUNK_A = NB_A // BPC
ROWS_PER_TILE = N_PAD // 16
ZROWS = 200
N_PAD_C = 50048
NB_C = N_PAD_C // BATCH
MM_BLK = 2048
NG = 3

_SPLAT_DNUMS = lax.GatherDimensionNumbers(
    offset_dims=(), collapsed_slice_dims=(0,), start_index_map=(0,))


def _lane_splat(vec16, lane):
    idx = jnp.full((16, 1), lane, jnp.int32)
    return lax.gather(vec16, idx, dimension_numbers=_SPLAT_DNUMS,
                      slice_sizes=(1,),
                      mode=lax.GatherScatterMode.PROMISE_IN_BOUNDS)


def _pool_body(xf, rowp, colp, valp, pooled,
               spmem, rows0, rows1, col7, col7t, row7, val7, zbuf,
               sem0, sem1):
    c = lax.axis_index("c")
    t = lax.axis_index("s")
    rows_ring = (rows0, rows1)
    sem_ring = (sem0, sem1)

    zero16 = jnp.zeros((16,), jnp.float32)

    def _zb(i, carry):
        zbuf[i, pl.ds(0, 16)] = zero16
        zbuf[i, pl.ds(16, 16)] = zero16
        return carry

    lax.fori_loop(0, ZROWS, _zb, 0)

    def _scale(rows_ref, j):
        def _grp(g, carry):
            vv = val7[j, pl.ds(g * 16, 16)]
            for i16 in range(16):
                v = _lane_splat(vv, i16)
                i = g * 16 + i16
                rows_ref[i, pl.ds(0, 16)] = rows_ref[i, pl.ds(0, 16)] * v
                rows_ref[i, pl.ds(16, 16)] = rows_ref[i, pl.ds(16, 16)] * v
            return carry

        lax.fori_loop(0, 8, _grp, 0)

    for p in range(2):
        for z in range(ROWS_PER_TILE // ZROWS):
            pltpu.sync_copy(
                zbuf, spmem.at[pl.ds(t * ROWS_PER_TILE + z * ZROWS, ZROWS)])
        plsc.subcore_barrier()

        for h in range(2):
            q = 2 * h + p

            @pl.when(c == h)
            def _scatter(q=q):
                def _chunk(ck, carry):
                    brow = t * NB_A + ck * BPC
                    pltpu.sync_copy(colp.at[pl.ds(brow, BPC)], col7)
                    pltpu.sync_copy(rowp.at[pl.ds(brow, BPC)], row7)
                    pltpu.sync_copy(valp.at[pl.ds(brow, BPC)], val7)
                    for j in range(BPC):
                        def _xf(g, carry2, j=j):
                            sl = pl.ds(g * 16, 16)
                            col7t[j, sl] = col7[j, sl] * 4 + q
                            return carry2

                        lax.fori_loop(0, 8, _xf, 0)
                    descs = {0: pltpu.async_copy(
                        xf.at[col7t.at[0]], rows_ring[0], sem_ring[0])}
                    for j in range(BPC):
                        if j + 1 < BPC:
                            descs[j + 1] = pltpu.async_copy(
                                xf.at[col7t.at[j + 1]],
                                rows_ring[(j + 1) % 2], sem_ring[(j + 1) % 2])
                        descs[j].wait()
                        _scale(rows_ring[j % 2], j)
                        pltpu.sync_copy(rows_ring[j % 2],
                                        spmem.at[row7.at[j]], add=True)
                    return carry

                lax.fori_loop(0, NCHUNK_A, _chunk, 0)

        plsc.subcore_barrier()

        for h in range(2):
            q = 2 * h + p

            @pl.when(c == h)
            def _copy_out(q=q):
                sl = pl.ds(t * ROWS_PER_TILE, ROWS_PER_TILE)
                pltpu.sync_copy(spmem.at[sl],
                                pooled.at[sl, pl.ds(CC * q, CC)])

        plsc.subcore_barrier()


def _matmul_body(xr, w, *outs):
    acc = lax.dot_general(xr[...], w[...], (((1,), (0,)), ((), ())),
                          preferred_element_type=jnp.float32)
    for g in range(NG):
        outs[g][...] = acc[:, C_IN * g:C_IN * (g + 1)]


def _spiral_body(st3, yf0, yf1, yf2, bias, out,
                 idxa, idxb,
                 ga0, ga1, ga2, ga3, ga4, ga5, ga6, ga7, ga8,
                 gb0, gb1, gb2, gb3, gb4, gb5, gb6, gb7, gb8,
                 obuf, bvm, sema, semb):
    c = lax.axis_index("c")
    s = lax.axis_index("s")
    wid = s * 2 + c
    yfs = (yf0, yf1, yf2)
    ga = (ga0, ga1, ga2, ga3, ga4, ga5, ga6, ga7, ga8)
    gb = (gb0, gb1, gb2, gb3, gb4, gb5, gb6, gb7, gb8)

    pltpu.sync_copy(bias, bvm)
    b0 = bvm[pl.ds(0, 16)]
    b1 = bvm[pl.ds(16, 16)]

    def _fire(kb, idx, g, sem):
        pltpu.sync_copy(st3.at[kb], idx)
        for j in range(L):
            def _xf(gg, carry, j=j):
                sl = pl.ds(gg * 16, 16)
                idx[j, sl] = idx[j, sl] * 4 + (j % 4)
                return carry

            lax.fori_loop(0, 8, _xf, 0)
        return [pltpu.async_copy(yfs[j // 4].at[idx.at[j]], g[j], sem)
                for j in range(L)]

    def _compute_store(kb, g):
        def _node(i, ncarry):
            for k in range(2):
                sl = pl.ds(16 * k, 16)
                acc = g[0][i, sl]
                for j in range(1, L):
                    acc = acc + g[j][i, sl]
                acc = acc + (b0 if k == 0 else b1)
                e = jnp.exp(acc) - 1.0
                obuf[i, sl] = jnp.where(acc > 0.0, acc, e)
            return ncarry

        lax.fori_loop(0, BATCH, _node, 0)
        pltpu.sync_copy(obuf, out.at[pl.ds(kb * BATCH, BATCH)])

    count = (NB_C - wid + 31) // 32
    cnt2 = count // 2

    def _pair(m, carry):
        kb_a = wid + (2 * m) * 32
        kb_b = wid + (2 * m + 1) * 32
        da = _fire(kb_a, idxa, ga, sema)
        db = _fire(kb_b, idxb, gb, semb)
        for d in da:
            d.wait()
        _compute_store(kb_a, ga)
        for d in db:
            d.wait()
        _compute_store(kb_b, gb)
        return carry

    lax.fori_loop(0, cnt2, _pair, 0)

    @pl.when(count % 2 == 1)
    def _tail():
        kb = wid + (2 * cnt2) * 32
        da = _fire(kb, idxa, ga, sema)
        for d in da:
            d.wait()
        _compute_store(kb, ga)


def _pool_call(xf, rowp2, colp2, valp2):
    mesh = plsc.VectorSubcoreMesh(core_axis_name="c", subcore_axis_name="s")
    f = pl.kernel(
        _pool_body,
        out_type=jax.ShapeDtypeStruct((N_PAD, C_IN), jnp.float32),
        mesh=mesh,
        scratch_types=[
            pltpu.VMEM_SHARED((N_PAD, CC), jnp.float32),
            pltpu.VMEM((BATCH, CC), jnp.float32),
            pltpu.VMEM((BATCH, CC), jnp.float32),
            pltpu.VMEM((BPC, BATCH), jnp.int32),
            pltpu.VMEM((BPC, BATCH), jnp.int32),
            pltpu.VMEM((BPC, BATCH), jnp.int32),
            pltpu.VMEM((BPC, BATCH), jnp.float32),
            pltpu.VMEM((ZROWS, CC), jnp.float32),
            pltpu.SemaphoreType.DMA,
            pltpu.SemaphoreType.DMA,
        ],
        compiler_params=pltpu.CompilerParams(use_tc_tiling_on_sc=False),
    )
    return f(xf, rowp2, colp2, valp2)


def _matmul_call(pooled, w_pad):
    grid = (N_PAD // MM_BLK,)
    in_specs = [pl.BlockSpec((MM_BLK, C_IN), lambda i: (i, 0)),
                pl.BlockSpec((C_IN, NG * C_IN), lambda i: (0, 0))]
    out_specs = [pl.BlockSpec((MM_BLK, C_IN), lambda i: (i, 0))
                 for _ in range(NG)]
    return pl.pallas_call(
        _matmul_body,
        grid=grid,
        in_specs=in_specs,
        out_specs=out_specs,
        out_shape=[jax.ShapeDtypeStruct((N_PAD, C_IN), jnp.float32)
                   for _ in range(NG)],
    )(pooled, w_pad)


def _spiral_call(st3, yfs, bias):
    mesh = plsc.VectorSubcoreMesh(core_axis_name="c", subcore_axis_name="s")
    f = pl.kernel(
        _spiral_body,
        out_type=jax.ShapeDtypeStruct((N_PAD_C, C_OUT), jnp.float32),
        mesh=mesh,
        scratch_types=(
            [pltpu.VMEM((L, BATCH), jnp.int32) for _ in range(2)]
            + [pltpu.VMEM((BATCH, C_OUT), jnp.float32) for _ in range(2 * L)]
            + [
                pltpu.VMEM((BATCH, C_OUT), jnp.float32),
                pltpu.VMEM((C_OUT,), jnp.float32),
                pltpu.SemaphoreType.DMA,
                pltpu.SemaphoreType.DMA,
            ]
        ),
        compiler_params=pltpu.CompilerParams(use_tc_tiling_on_sc=False),
    )
    return f(st3, *yfs, bias)


def kernel(x, trans_row, trans_col, trans_val, spiral_indices, W, b):
    xf = x.reshape(N_IN * NQ, CC)

    pad = NNZ_PAD - NNZ
    rowp2 = jnp.pad(trans_row.astype(jnp.int32), (0, pad)).reshape(-1, BATCH)
    colp2 = jnp.pad(trans_col.astype(jnp.int32), (0, pad)).reshape(-1, BATCH)
    valp2 = jnp.pad(trans_val, (0, pad)).reshape(-1, BATCH)

    st3 = jnp.pad(spiral_indices.astype(jnp.int32),
                  ((0, N_PAD_C - N_OUT), (0, 0)))
    st3 = st3.T.reshape(L, NB_C, BATCH).transpose(1, 0, 2)

    w_perm = W.reshape(L, C_IN, C_OUT).transpose(1, 0, 2).reshape(
        C_IN, L * C_OUT)
    w_pad = jnp.pad(w_perm, ((0, 0), (0, NG * C_IN - L * C_OUT)))

    pooled = _pool_call(xf, rowp2, colp2, valp2)
    ys = _matmul_call(pooled, w_pad)
    yfs = [y.reshape(NQ * N_PAD, CC) for y in ys]
    out = _spiral_call(st3, yfs, b)

    return out[:N_OUT].reshape(1, N_OUT, C_OUT)
